# baseline (speedup 1.0000x reference)
import jax
import jax.numpy as jnp
from jax.experimental import pallas as pl

N_BEHAVIORS = 8
_BB = 64


def _body(pb_ref, t_ref, beh_ref, pad_ref, s_ref, hp_ref, hn_ref):
    idx = jnp.maximum(beh_ref[...] - 1, 0)
    pref = jnp.zeros_like(pad_ref[...])
    for k in range(N_BEHAVIORS):
        pref = pref + jnp.where(idx == k, pb_ref[0, k], 0.0)
    t = jax.nn.sigmoid(t_ref[...])
    pad = pad_ref[...]
    mask = jnp.where(pref - t > 0.0, pad, 0.0)
    neg = (1.0 - mask) * pad
    s = s_ref[...]
    hp_ref[...] = s * mask[:, :, None]
    hn_ref[...] = s * neg[:, :, None]


def kernel(S, behavior_seq, padding_mask, lambda_raw, threshold):
    B, L, D = S.shape
    lam = jax.nn.softplus(lambda_raw) + 1e-06
    log_pmf = -lam + lam * jnp.log(lam) - jax.lax.lgamma(lam + 1.0)
    p_b = (jnp.exp(log_pmf) + 1.0).reshape(1, N_BEHAVIORS)
    t = threshold[:L].reshape(1, L)

    grid = (B // _BB,)
    hp, hn = pl.pallas_call(
        _body,
        grid=grid,
        in_specs=[
            pl.BlockSpec((1, N_BEHAVIORS), lambda i: (0, 0)),
            pl.BlockSpec((1, L), lambda i: (0, 0)),
            pl.BlockSpec((_BB, L), lambda i: (i, 0)),
            pl.BlockSpec((_BB, L), lambda i: (i, 0)),
            pl.BlockSpec((_BB, L, D), lambda i: (i, 0, 0)),
        ],
        out_specs=[
            pl.BlockSpec((_BB, L, D), lambda i: (i, 0, 0)),
            pl.BlockSpec((_BB, L, D), lambda i: (i, 0, 0)),
        ],
        out_shape=[
            jax.ShapeDtypeStruct((B, L, D), jnp.float32),
            jax.ShapeDtypeStruct((B, L, D), jnp.float32),
        ],
    )(p_b, t, behavior_seq, padding_mask, S)
    return (hp, hn)


# speedup vs baseline: 1.6315x; 1.6315x over previous
import jax
import jax.numpy as jnp
from jax.experimental import pallas as pl

N_BEHAVIORS = 8
_BB = 128


def _body(pb_ref, t_ref, r_ref, beh_ref, pad_ref, s_ref, hp_ref, hn_ref):
    bb = beh_ref.shape[0]
    L = beh_ref.shape[1]
    idx = jnp.maximum(beh_ref[...] - 1, 0)
    pref = jnp.zeros((bb, L), jnp.float32)
    for k in range(N_BEHAVIORS):
        pref = pref + jnp.where(idx == k, pb_ref[0, k], 0.0)
    t = jax.nn.sigmoid(t_ref[...])
    pad = pad_ref[...]
    m = jnp.where(pref - t > 0.0, pad, 0.0)
    neg = (1.0 - m) * pad
    r = r_ref[...]
    mexp = jax.lax.dot_general(m, r, (((1,), (0,)), ((), ())),
                               preferred_element_type=jnp.float32)
    negexp = jax.lax.dot_general(neg, r, (((1,), (0,)), ((), ())),
                                 preferred_element_type=jnp.float32)
    s = s_ref[...]
    hp_ref[...] = s * mexp
    hn_ref[...] = s * negexp


def kernel(S, behavior_seq, padding_mask, lambda_raw, threshold):
    B, L, D = S.shape
    lam = jax.nn.softplus(lambda_raw) + 1e-06
    log_pmf = -lam + lam * jnp.log(lam) - jax.lax.lgamma(lam + 1.0)
    p_b = (jnp.exp(log_pmf) + 1.0).reshape(1, N_BEHAVIORS)
    t = threshold[:L].reshape(1, L)
    S2 = S.reshape(B, L * D)
    R = (jax.lax.broadcasted_iota(jnp.int32, (L, L * D), 1) // D
         == jax.lax.broadcasted_iota(jnp.int32, (L, L * D), 0)
         ).astype(jnp.float32)

    grid = (B // _BB,)
    hp, hn = pl.pallas_call(
        _body,
        grid=grid,
        in_specs=[
            pl.BlockSpec((1, N_BEHAVIORS), lambda i: (0, 0)),
            pl.BlockSpec((1, L), lambda i: (0, 0)),
            pl.BlockSpec((L, L * D), lambda i: (0, 0)),
            pl.BlockSpec((_BB, L), lambda i: (i, 0)),
            pl.BlockSpec((_BB, L), lambda i: (i, 0)),
            pl.BlockSpec((_BB, L * D), lambda i: (i, 0)),
        ],
        out_specs=[
            pl.BlockSpec((_BB, L * D), lambda i: (i, 0)),
            pl.BlockSpec((_BB, L * D), lambda i: (i, 0)),
        ],
        out_shape=[
            jax.ShapeDtypeStruct((B, L * D), jnp.float32),
            jax.ShapeDtypeStruct((B, L * D), jnp.float32),
        ],
    )(p_b, t, R, behavior_seq, padding_mask, S2)
    return (hp.reshape(B, L, D), hn.reshape(B, L, D))


# speedup vs baseline: 6.0684x; 3.7197x over previous
import jax
import jax.numpy as jnp
from jax.experimental import pallas as pl

N_BEHAVIORS = 8
_BB = 128


def _body(pb_ref, t_ref, beh_ref, pad_ref, s_ref, hp_ref, hn_ref):
    bb = beh_ref.shape[0]
    L = beh_ref.shape[1]
    idx = jnp.maximum(beh_ref[...] - 1, 0)
    pref = jnp.zeros((bb, L), jnp.float32)
    for k in range(N_BEHAVIORS):
        pref = pref + jnp.where(idx == k, pb_ref[0, k], 0.0)
    t = jax.nn.sigmoid(t_ref[...])
    pad = pad_ref[...]
    m = jnp.where(pref - t > 0.0, pad, 0.0)
    neg = (1.0 - m) * pad
    mT = m.T[:, None, :]
    negT = neg.T[:, None, :]
    s = s_ref[...]
    hp_ref[...] = s * mT
    hn_ref[...] = s * negT


def kernel(S, behavior_seq, padding_mask, lambda_raw, threshold):
    B, L, D = S.shape
    lam = jax.nn.softplus(lambda_raw) + 1e-06
    log_pmf = -lam + lam * jnp.log(lam) - jax.lax.lgamma(lam + 1.0)
    p_b = (jnp.exp(log_pmf) + 1.0).reshape(1, N_BEHAVIORS)
    t = threshold[:L].reshape(1, L)
    St = S.transpose(1, 2, 0)

    grid = (B // _BB,)
    hp, hn = pl.pallas_call(
        _body,
        grid=grid,
        in_specs=[
            pl.BlockSpec((1, N_BEHAVIORS), lambda i: (0, 0)),
            pl.BlockSpec((1, L), lambda i: (0, 0)),
            pl.BlockSpec((_BB, L), lambda i: (i, 0)),
            pl.BlockSpec((_BB, L), lambda i: (i, 0)),
            pl.BlockSpec((L, D, _BB), lambda i: (0, 0, i)),
        ],
        out_specs=[
            pl.BlockSpec((L, D, _BB), lambda i: (0, 0, i)),
            pl.BlockSpec((L, D, _BB), lambda i: (0, 0, i)),
        ],
        out_shape=[
            jax.ShapeDtypeStruct((L, D, B), jnp.float32),
            jax.ShapeDtypeStruct((L, D, B), jnp.float32),
        ],
    )(p_b, t, behavior_seq, padding_mask, St)
    return (hp.transpose(2, 0, 1), hn.transpose(2, 0, 1))


# speedup vs baseline: 6.3029x; 1.0386x over previous
import jax
import jax.numpy as jnp
from jax.experimental import pallas as pl

N_BEHAVIORS = 8
_BB = 128


def _body(pb_ref, t_ref, beh_ref, pad_ref, s_ref, hp_ref, hn_ref):
    L, bb = beh_ref.shape
    idx = jnp.maximum(beh_ref[...] - 1, 0)
    pref = jnp.zeros((L, bb), jnp.float32)
    for k in range(N_BEHAVIORS):
        pref = pref + jnp.where(idx == k, pb_ref[0, k], 0.0)
    pad = pad_ref[...]
    m = jnp.where(pref - t_ref[...] > 0.0, pad, 0.0)
    neg = (1.0 - m) * pad
    s = s_ref[...]
    hp_ref[...] = s * m[:, None, :]
    hn_ref[...] = s * neg[:, None, :]


def kernel(S, behavior_seq, padding_mask, lambda_raw, threshold):
    B, L, D = S.shape
    lam = jax.nn.softplus(lambda_raw) + 1e-06
    log_pmf = -lam + lam * jnp.log(lam) - jax.lax.lgamma(lam + 1.0)
    p_b = (jnp.exp(log_pmf) + 1.0).reshape(1, N_BEHAVIORS)
    t = jax.nn.sigmoid(threshold[:L]).reshape(L, 1)
    St = S.transpose(1, 2, 0)
    behT = behavior_seq.T
    padT = padding_mask.T

    grid = (B // _BB,)
    hp, hn = pl.pallas_call(
        _body,
        grid=grid,
        in_specs=[
            pl.BlockSpec((1, N_BEHAVIORS), lambda i: (0, 0)),
            pl.BlockSpec((L, 1), lambda i: (0, 0)),
            pl.BlockSpec((L, _BB), lambda i: (0, i)),
            pl.BlockSpec((L, _BB), lambda i: (0, i)),
            pl.BlockSpec((L, D, _BB), lambda i: (0, 0, i)),
        ],
        out_specs=[
            pl.BlockSpec((L, D, _BB), lambda i: (0, 0, i)),
            pl.BlockSpec((L, D, _BB), lambda i: (0, 0, i)),
        ],
        out_shape=[
            jax.ShapeDtypeStruct((L, D, B), jnp.float32),
            jax.ShapeDtypeStruct((L, D, B), jnp.float32),
        ],
    )(p_b, t, behT, padT, St)
    return (hp.transpose(2, 0, 1), hn.transpose(2, 0, 1))


# speedup vs baseline: 6.3559x; 1.0084x over previous
import jax
import jax.numpy as jnp
from jax.experimental import pallas as pl

N_BEHAVIORS = 8
_LB = 8


def _body(pb_ref, t_ref, beh_ref, pad_ref, s_ref, hp_ref, hn_ref):
    lb, bb = beh_ref.shape
    idx = jnp.maximum(beh_ref[...] - 1, 0)
    pref = jnp.zeros((lb, bb), jnp.float32)
    for k in range(N_BEHAVIORS):
        pref = pref + jnp.where(idx == k, pb_ref[0, k], 0.0)
    pad = pad_ref[...]
    m = jnp.where(pref - t_ref[...] > 0.0, pad, 0.0)
    neg = (1.0 - m) * pad
    s = s_ref[...]
    hp_ref[...] = s * m[:, None, :]
    hn_ref[...] = s * neg[:, None, :]


def kernel(S, behavior_seq, padding_mask, lambda_raw, threshold):
    B, L, D = S.shape
    lam = jax.nn.softplus(lambda_raw) + 1e-06
    log_pmf = -lam + lam * jnp.log(lam) - jax.lax.lgamma(lam + 1.0)
    p_b = (jnp.exp(log_pmf) + 1.0).reshape(1, N_BEHAVIORS)
    t = jax.nn.sigmoid(threshold[:L]).reshape(L, 1)
    St = S.transpose(1, 2, 0)
    behT = behavior_seq.T
    padT = padding_mask.T

    grid = (L // _LB,)
    hp, hn = pl.pallas_call(
        _body,
        grid=grid,
        in_specs=[
            pl.BlockSpec((1, N_BEHAVIORS), lambda i: (0, 0)),
            pl.BlockSpec((_LB, 1), lambda i: (i, 0)),
            pl.BlockSpec((_LB, B), lambda i: (i, 0)),
            pl.BlockSpec((_LB, B), lambda i: (i, 0)),
            pl.BlockSpec((_LB, D, B), lambda i: (i, 0, 0)),
        ],
        out_specs=[
            pl.BlockSpec((_LB, D, B), lambda i: (i, 0, 0)),
            pl.BlockSpec((_LB, D, B), lambda i: (i, 0, 0)),
        ],
        out_shape=[
            jax.ShapeDtypeStruct((L, D, B), jnp.float32),
            jax.ShapeDtypeStruct((L, D, B), jnp.float32),
        ],
    )(p_b, t, behT, padT, St)
    return (hp.transpose(2, 0, 1), hn.transpose(2, 0, 1))
